# Initial kernel scaffold; baseline (speedup 1.0000x reference)
import jax
import jax.numpy as jnp
from jax.experimental import pallas as pl


def kernel(texts, embed, W1, b1, W2, b2):
    raise NotImplementedError("write your pallas kernel here")


# speedup vs baseline: 11.9541x; 11.9541x over previous
import functools

import jax
import jax.numpy as jnp
from jax import lax
from jax.experimental import pallas as pl
from jax.experimental.pallas import tpu as pltpu
from jax.experimental.pallas import tpu_sc as plsc

B = 4096
S = 200
D = 128
H = 512
VOUT = 1000
VPAD = 1024

S0 = 104
S1 = S - S0

_NC = 2
_NS = 16
NW = _NC * _NS
BPW = B // NW


def _pool_sc(texts, embed):
    mesh = plsc.VectorSubcoreMesh(core_axis_name="c", subcore_axis_name="s")

    @functools.partial(
        pl.kernel,
        out_type=jax.ShapeDtypeStruct((B, D), jnp.float32),
        mesh=mesh,
        scratch_types=[
            pltpu.VMEM((BPW * S,), jnp.int32),
            pltpu.VMEM((S, D), jnp.float32),
            pltpu.VMEM((S, D), jnp.float32),
            pltpu.VMEM((BPW, D), jnp.float32),
            pltpu.SemaphoreType.DMA,
            pltpu.SemaphoreType.DMA,
        ],
    )
    def k(texts_hbm, embed_hbm, out_hbm, idx_v, rows_a, rows_b, out_v,
          sem_a, sem_b):
        wid = lax.axis_index("s") * _NC + lax.axis_index("c")
        base = wid * BPW

        pltpu.sync_copy(texts_hbm.at[pl.ds(base * S, BPW * S)], idx_v)

        def fire(b, rows_ref, sem):
            off = pl.multiple_of(b * S, 8)
            pltpu.async_copy(
                embed_hbm.at[idx_v.at[pl.ds(off, S0)]],
                rows_ref.at[pl.ds(0, S0)], sem)
            pltpu.async_copy(
                embed_hbm.at[idx_v.at[pl.ds(off + S0, S1)]],
                rows_ref.at[pl.ds(S0, S1)], sem)

        def wait(rows_ref, sem):
            pltpu.make_async_copy(
                embed_hbm.at[pl.ds(0, S)], rows_ref, sem).wait()

        scale = jnp.float32(1.0 / S)

        def accum_store(b, rows_ref):
            def body(s, accs):
                return tuple(accs[j] + rows_ref[s, pl.ds(j * 16, 16)]
                             for j in range(8))
            accs = tuple(jnp.zeros((16,), jnp.float32) for _ in range(8))
            accs = lax.fori_loop(0, S, body, accs)
            for j in range(8):
                out_v[b, pl.ds(j * 16, 16)] = accs[j] * scale

        fire(0, rows_a, sem_a)

        def loop_body(i, carry):
            b = i * 2
            fire(b + 1, rows_b, sem_b)
            wait(rows_a, sem_a)
            accum_store(b, rows_a)

            @pl.when(i < BPW // 2 - 1)
            def _():
                fire(b + 2, rows_a, sem_a)

            wait(rows_b, sem_b)
            accum_store(b + 1, rows_b)
            return carry

        lax.fori_loop(0, BPW // 2, loop_body, 0)
        pltpu.sync_copy(out_v, out_hbm.at[pl.ds(base, BPW)])

    return k(texts.reshape(B * S), embed)


def _mlp_body(p_ref, w1_ref, b1_ref, w2_ref, b2_ref, o_ref):
    h = jnp.dot(p_ref[...], w1_ref[...],
                preferred_element_type=jnp.float32) + b1_ref[...]
    h = jnp.maximum(h, 0.0)
    o_ref[...] = jnp.dot(h, w2_ref[...],
                         preferred_element_type=jnp.float32) + b2_ref[...]


def _mlp_tc(pooled, W1, b1, W2, b2):
    BM = 512
    W2p = jnp.pad(W2, ((0, 0), (0, VPAD - VOUT)))
    b2p = jnp.pad(b2, (0, VPAD - VOUT))
    out = pl.pallas_call(
        _mlp_body,
        grid=(B // BM,),
        in_specs=[
            pl.BlockSpec((BM, D), lambda i: (i, 0)),
            pl.BlockSpec((D, H), lambda i: (0, 0)),
            pl.BlockSpec((1, H), lambda i: (0, 0)),
            pl.BlockSpec((H, VPAD), lambda i: (0, 0)),
            pl.BlockSpec((1, VPAD), lambda i: (0, 0)),
        ],
        out_specs=pl.BlockSpec((BM, VPAD), lambda i: (i, 0)),
        out_shape=jax.ShapeDtypeStruct((B, VPAD), jnp.float32),
    )(pooled, W1, b1.reshape(1, H), W2p, b2p.reshape(1, VPAD))
    return out[:, :VOUT]


def kernel(texts, embed, W1, b1, W2, b2):
    pooled = _pool_sc(texts, embed)
    return _mlp_tc(pooled, W1, b1, W2, b2)


# speedup vs baseline: 13.2099x; 1.1050x over previous
import functools

import jax
import jax.numpy as jnp
from jax import lax
from jax.experimental import pallas as pl
from jax.experimental.pallas import tpu as pltpu
from jax.experimental.pallas import tpu_sc as plsc

B = 4096
S = 200
D = 128
H = 512
VOUT = 1000

_NC = 2
_NS = 16
NW = _NC * _NS
BPW = B // NW
PAIRS = BPW // 2

_SPLITS = ((0, 104), (104, 104), (208, 104), (312, 88))


def _pool_sc(texts, embed):
    mesh = plsc.VectorSubcoreMesh(core_axis_name="c", subcore_axis_name="s")

    @functools.partial(
        pl.kernel,
        out_type=jax.ShapeDtypeStruct((B * D,), jnp.float32),
        mesh=mesh,
        scratch_types=[
            pltpu.VMEM((BPW * S,), jnp.int32),
            pltpu.VMEM((2 * S, D), jnp.float32),
            pltpu.VMEM((2 * S, D), jnp.float32),
            pltpu.VMEM((256,), jnp.float32),
            pltpu.SemaphoreType.DMA,
            pltpu.SemaphoreType.DMA,
            pltpu.SemaphoreType.DMA,
        ],
    )
    def k(texts_hbm, embed_hbm, out_hbm, idx_v, rows_a, rows_b, stage_v,
          sem_a, sem_b, sem_o):
        wid = lax.axis_index("s") * _NC + lax.axis_index("c")
        base = wid * BPW

        pltpu.sync_copy(texts_hbm.at[pl.ds(base * S, BPW * S)], idx_v)

        def fire_pair(p, rows_ref, sem):
            off = pl.multiple_of(p * 2 * S, 16)
            for o, n in _SPLITS:
                pltpu.async_copy(
                    embed_hbm.at[idx_v.at[pl.ds(off + o, n)]],
                    rows_ref.at[pl.ds(o, n)], sem)

        def wait_pair(rows_ref, sem):
            pltpu.make_async_copy(
                embed_hbm.at[pl.ds(0, 2 * S)], rows_ref, sem).wait()

        scale = jnp.float32(1.0 / S)

        def drain_out():
            pltpu.make_async_copy(
                out_hbm.at[pl.ds(0, D)], stage_v.at[pl.ds(0, D)],
                sem_o).wait()

        def accum_out(b, r0, rows_ref):
            def body(s, accs):
                return tuple(accs[j] + rows_ref[s, pl.ds(j * 16, 16)]
                             for j in range(8))
            accs = lax.fori_loop(
                r0, r0 + S, body,
                tuple(jnp.zeros((16,), jnp.float32) for _ in range(8)))
            slot = (b % 2) * D

            @pl.when(b >= 2)
            def _():
                drain_out()

            for j in range(8):
                stage_v[pl.ds(slot + j * 16, 16)] = accs[j] * scale
            pltpu.async_copy(
                stage_v.at[pl.ds(slot, D)],
                out_hbm.at[pl.ds((base + b) * D, D)], sem_o)

        fire_pair(0, rows_a, sem_a)
        fire_pair(1, rows_b, sem_b)

        def loop_body(i, carry):
            p = i * 2
            b = p * 2
            wait_pair(rows_a, sem_a)
            accum_out(b, 0, rows_a)
            accum_out(b + 1, S, rows_a)

            @pl.when(p + 2 < PAIRS)
            def _():
                fire_pair(p + 2, rows_a, sem_a)

            wait_pair(rows_b, sem_b)
            accum_out(b + 2, 0, rows_b)
            accum_out(b + 3, S, rows_b)

            @pl.when(p + 3 < PAIRS)
            def _():
                fire_pair(p + 3, rows_b, sem_b)

            return carry

        lax.fori_loop(0, PAIRS // 2, loop_body, 0)
        drain_out()
        drain_out()

    return k(texts.reshape(B * S), embed)


def _mlp_body(p_ref, w1_ref, b1_ref, w2_ref, b2_ref, o_ref):
    h = jnp.dot(p_ref[...], w1_ref[...],
                preferred_element_type=jnp.float32) + b1_ref[...]
    h = jnp.maximum(h, 0.0)
    o_ref[...] = jnp.dot(h, w2_ref[...],
                         preferred_element_type=jnp.float32) + b2_ref[...]


def _mlp_tc(pooled, W1, b1, W2, b2):
    BM = 512
    return pl.pallas_call(
        _mlp_body,
        grid=(B // BM,),
        in_specs=[
            pl.BlockSpec((BM, D), lambda i: (i, 0)),
            pl.BlockSpec((D, H), lambda i: (0, 0)),
            pl.BlockSpec((1, H), lambda i: (0, 0)),
            pl.BlockSpec((H, VOUT), lambda i: (0, 0)),
            pl.BlockSpec((1, VOUT), lambda i: (0, 0)),
        ],
        out_specs=pl.BlockSpec((BM, VOUT), lambda i: (i, 0)),
        out_shape=jax.ShapeDtypeStruct((B, VOUT), jnp.float32),
    )(pooled, W1, b1.reshape(1, H), W2, b2.reshape(1, VOUT))


def kernel(texts, embed, W1, b1, W2, b2):
    pooled = _pool_sc(texts, embed).reshape(B, D)
    return _mlp_tc(pooled, W1, b1, W2, b2)
